# baseline (speedup 1.0000x reference)
import jax
import jax.numpy as jnp
from jax.experimental import pallas as pl
from jax.experimental.pallas import tpu as pltpu

def _h_kernel(adj_ref, x_ref, w_ref, b_ref, h_ref):
    t = jnp.dot(adj_ref[...], x_ref[...], preferred_element_type=jnp.float32)
    z = jnp.dot(t, w_ref[...], preferred_element_type=jnp.float32) + b_ref[...]
    h_ref[...] = jnp.maximum(z, jnp.float32(0.0))

def kernel(x, adj, weight, bias):
    n, nhid = x.shape
    tm = 1024
    bias2d = bias.reshape(1, nhid)
    h = pl.pallas_call(
        _h_kernel,
        out_shape=jax.ShapeDtypeStruct((n, nhid), jnp.float32),
        grid=(n // tm,),
        in_specs=[
            pl.BlockSpec((tm, n), lambda i: (i, 0)),
            pl.BlockSpec((n, nhid), lambda i: (0, 0)),
            pl.BlockSpec((nhid, nhid), lambda i: (0, 0)),
            pl.BlockSpec((1, nhid), lambda i: (0, 0)),
        ],
        out_specs=pl.BlockSpec((tm, nhid), lambda i: (i, 0)),
        compiler_params=pltpu.CompilerParams(
            dimension_semantics=("parallel",),
            vmem_limit_bytes=56 * 1024 * 1024,
        ),
    )(adj, x, weight, bias2d)
    return h
